# baseline (speedup 1.0000x reference)
import jax
import jax.numpy as jnp
from jax.experimental import pallas as pl
from jax.experimental.pallas import tpu as pltpu

N = 10000
D = 128
H = 64
G = 64
O = 2
BI = 200
NI = N // BI
OP = 128


def _fused_kernel(bounds_ref, x_ref, adj_l_ref, batch_ref, Wg_ref,
                  bg_ref, W1t_ref, b1_ref, W2t_ref, b2_ref, Wot_ref, bot_ref,
                  out_ref, support_ref, p_ref):
    i = pl.program_id(0)

    @pl.when(i == 0)
    def _init():
        support_ref[...] = jnp.dot(x_ref[...], Wg_ref[...],
                                   preferred_element_type=jnp.float32)
        p_ref[...] = jnp.full((G, H), -jnp.inf, dtype=jnp.float32)

    s = support_ref[...]
    ht = jnp.dot(adj_l_ref[...], s, preferred_element_type=jnp.float32)
    ht = ht + bg_ref[...]
    ht = jnp.where(ht >= 0, ht, 0.01 * ht)

    ids = batch_ref[0]

    def _seg_body(g, carry):
        red = jnp.max(jnp.where(ids == g, ht, -jnp.inf), axis=0, keepdims=True)
        p_ref[pl.ds(g, 1), :] = jnp.maximum(p_ref[pl.ds(g, 1), :], red)
        return carry

    jax.lax.fori_loop(bounds_ref[i, 0], bounds_ref[i, 1] + 1, _seg_body, 0)

    @pl.when(i == NI - 1)
    def _head():
        p = p_ref[...]
        z = jnp.dot(p, W1t_ref[...], preferred_element_type=jnp.float32)
        z = z + b1_ref[...]
        z = jnp.where(z >= 0, z, 0.01 * z)
        z = jnp.dot(z, W2t_ref[...], preferred_element_type=jnp.float32)
        z = z + b2_ref[...]
        z = jnp.where(z >= 0, z, 0.01 * z)
        out_ref[...] = jnp.dot(z, Wot_ref[...],
                               preferred_element_type=jnp.float32) + bot_ref[...]


def kernel(x, adj, batch, n_nodes, Wg, bg, W1, b1, W2, b2, Wo, bo):
    del n_nodes
    batch3 = batch.reshape(NI, BI, 1)
    b2d = batch.reshape(NI, BI)
    bounds = jnp.stack([b2d[:, 0], b2d[:, -1]], axis=1)
    W1t = W1.T
    W2t = W2.T
    Wot = jnp.zeros((H, OP), jnp.float32).at[:, :O].set(Wo.T)
    bot = jnp.zeros((1, OP), jnp.float32).at[:, :O].set(bo)
    out = pl.pallas_call(
        _fused_kernel,
        grid=(NI,),
        in_specs=[
            pl.BlockSpec(memory_space=pltpu.SMEM),
            pl.BlockSpec((N, D), lambda i: (0, 0)),
            pl.BlockSpec((BI, N), lambda i: (i, 0)),
            pl.BlockSpec((1, BI, 1), lambda i: (i, 0, 0)),
            pl.BlockSpec((D, H), lambda i: (0, 0)),
            pl.BlockSpec((1, H), lambda i: (0, 0)),
            pl.BlockSpec((H, H), lambda i: (0, 0)),
            pl.BlockSpec((1, H), lambda i: (0, 0)),
            pl.BlockSpec((H, H), lambda i: (0, 0)),
            pl.BlockSpec((1, H), lambda i: (0, 0)),
            pl.BlockSpec((H, OP), lambda i: (0, 0)),
            pl.BlockSpec((1, OP), lambda i: (0, 0)),
        ],
        out_specs=pl.BlockSpec((G, OP), lambda i: (0, 0)),
        out_shape=jax.ShapeDtypeStruct((G, OP), jnp.float32),
        scratch_shapes=[
            pltpu.VMEM((N, H), jnp.float32),
            pltpu.VMEM((G, H), jnp.float32),
        ],
    )(bounds, x, adj, batch3, Wg, bg, W1t, b1, W2t, b2, Wot, bot)
    return out[:, :O]


# speedup vs baseline: 1.0760x; 1.0760x over previous
import jax
import jax.numpy as jnp
from jax import lax
from jax.experimental import pallas as pl
from jax.experimental.pallas import tpu as pltpu

N = 10000
D = 128
H = 64
G = 64
O = 2
BI = 400
NI = N // BI

_NT = (((1,), (1,)), ((), ()))


def _fused_kernel(bounds_ref, x_ref, adj_t_ref, adj_b_ref, batch_ref, Wg_ref,
                  bg_ref, W1_ref, b1_ref, W2_ref, b2_ref, Wo_ref, bo_ref,
                  out_ref, support_ref, p_ref):
    i = pl.program_id(0)

    @pl.when(i == 0)
    def _init():
        support_ref[...] = jnp.dot(x_ref[...], Wg_ref[...],
                                   preferred_element_type=jnp.float32)
        p_ref[...] = jnp.full((G, H), -jnp.inf, dtype=jnp.float32)

    s = support_ref[...]
    ht = jnp.dot(adj_t_ref[...], s, preferred_element_type=jnp.float32)
    hb = jnp.dot(adj_b_ref[...], s, preferred_element_type=jnp.float32)
    bgv = bg_ref[...]
    ht = ht + bgv
    hb = hb + bgv
    ht = jnp.where(ht >= 0, ht, 0.01 * ht)
    hb = jnp.where(hb >= 0, hb, 0.01 * hb)

    ids = batch_ref[0]
    ids_t = ids[: BI // 2, :]
    ids_b = ids[BI // 2 :, :]

    def _seg_body(g, carry):
        rt = jnp.max(jnp.where(ids_t == g, ht, -jnp.inf), axis=0, keepdims=True)
        rb = jnp.max(jnp.where(ids_b == g, hb, -jnp.inf), axis=0, keepdims=True)
        red = jnp.maximum(rt, rb)
        p_ref[pl.ds(g, 1), :] = jnp.maximum(p_ref[pl.ds(g, 1), :], red)
        return carry

    jax.lax.fori_loop(bounds_ref[i, 0], bounds_ref[i, 1] + 1, _seg_body, 0)

    @pl.when(i == NI - 1)
    def _head():
        p = p_ref[...]
        z = lax.dot_general(p, W1_ref[...], _NT,
                            preferred_element_type=jnp.float32) + b1_ref[...]
        z = jnp.where(z >= 0, z, 0.01 * z)
        z = lax.dot_general(z, W2_ref[...], _NT,
                            preferred_element_type=jnp.float32) + b2_ref[...]
        z = jnp.where(z >= 0, z, 0.01 * z)
        out_ref[...] = lax.dot_general(z, Wo_ref[...], _NT,
                                       preferred_element_type=jnp.float32) + bo_ref[...]


def kernel(x, adj, batch, n_nodes, Wg, bg, W1, b1, W2, b2, Wo, bo):
    del n_nodes
    batch3 = batch.reshape(NI, BI, 1)
    bounds = jnp.stack([batch[::BI], batch[BI - 1::BI]], axis=1)
    return pl.pallas_call(
        _fused_kernel,
        grid=(NI,),
        in_specs=[
            pl.BlockSpec(memory_space=pltpu.SMEM),
            pl.BlockSpec((N, D), lambda i: (0, 0)),
            pl.BlockSpec((BI // 2, N), lambda i: (2 * i, 0)),
            pl.BlockSpec((BI // 2, N), lambda i: (2 * i + 1, 0)),
            pl.BlockSpec((1, BI, 1), lambda i: (i, 0, 0)),
            pl.BlockSpec((D, H), lambda i: (0, 0)),
            pl.BlockSpec((1, H), lambda i: (0, 0)),
            pl.BlockSpec((H, H), lambda i: (0, 0)),
            pl.BlockSpec((1, H), lambda i: (0, 0)),
            pl.BlockSpec((H, H), lambda i: (0, 0)),
            pl.BlockSpec((1, H), lambda i: (0, 0)),
            pl.BlockSpec((O, H), lambda i: (0, 0)),
            pl.BlockSpec((1, O), lambda i: (0, 0)),
        ],
        out_specs=pl.BlockSpec((G, O), lambda i: (0, 0)),
        out_shape=jax.ShapeDtypeStruct((G, O), jnp.float32),
        scratch_shapes=[
            pltpu.VMEM((N, H), jnp.float32),
            pltpu.VMEM((G, H), jnp.float32),
        ],
    )(bounds, x, adj, adj, batch3, Wg, bg, W1, b1, W2, b2, Wo, bo)


# speedup vs baseline: 1.0898x; 1.0128x over previous
import jax
import jax.numpy as jnp
from jax import lax
from jax.experimental import pallas as pl
from jax.experimental.pallas import tpu as pltpu

N = 10000
D = 128
H = 64
G = 64
O = 2
BI = 400
NI = N // BI
SPLIT = 5
BS = BI // SPLIT

_NT = (((1,), (1,)), ((), ()))


def _issue_block(adj_ref, abuf_ref, sem, block, slot):
    base = block * BI
    for j in range(SPLIT):
        pltpu.make_async_copy(
            adj_ref.at[pl.ds(base + j * BS, BS), :],
            abuf_ref.at[slot, pl.ds(j * BS, BS), :],
            sem.at[slot, j],
        ).start()


def _wait_block(adj_ref, abuf_ref, sem, block, slot):
    base = block * BI
    for j in range(SPLIT):
        pltpu.make_async_copy(
            adj_ref.at[pl.ds(base + j * BS, BS), :],
            abuf_ref.at[slot, pl.ds(j * BS, BS), :],
            sem.at[slot, j],
        ).wait()


def _fused_kernel(bounds_ref, x_ref, adj_ref, batch_ref, Wg_ref,
                  bg_ref, W1_ref, b1_ref, W2_ref, b2_ref, Wo_ref, bo_ref,
                  out_ref, support_ref, p_ref, abuf_ref, sem):
    i = pl.program_id(0)
    slot = lax.rem(i, 2)

    @pl.when(i == 0)
    def _init():
        _issue_block(adj_ref, abuf_ref, sem, 0, 0)
        support_ref[...] = jnp.dot(x_ref[...], Wg_ref[...],
                                   preferred_element_type=jnp.float32)
        p_ref[...] = jnp.full((G, H), -jnp.inf, dtype=jnp.float32)

    @pl.when(i + 1 < NI)
    def _prefetch():
        _issue_block(adj_ref, abuf_ref, sem, i + 1, 1 - slot)

    _wait_block(adj_ref, abuf_ref, sem, i, slot)

    s = support_ref[...]
    ht = jnp.dot(abuf_ref[slot, : BI // 2, :], s,
                 preferred_element_type=jnp.float32)
    hb = jnp.dot(abuf_ref[slot, BI // 2 :, :], s,
                 preferred_element_type=jnp.float32)
    bgv = bg_ref[...]
    ht = ht + bgv
    hb = hb + bgv
    ht = jnp.where(ht >= 0, ht, 0.01 * ht)
    hb = jnp.where(hb >= 0, hb, 0.01 * hb)

    ids = batch_ref[0]
    ids_t = ids[: BI // 2, :]
    ids_b = ids[BI // 2 :, :]

    def _seg_body(g, carry):
        rt = jnp.max(jnp.where(ids_t == g, ht, -jnp.inf), axis=0, keepdims=True)
        rb = jnp.max(jnp.where(ids_b == g, hb, -jnp.inf), axis=0, keepdims=True)
        red = jnp.maximum(rt, rb)
        p_ref[pl.ds(g, 1), :] = jnp.maximum(p_ref[pl.ds(g, 1), :], red)
        return carry

    jax.lax.fori_loop(bounds_ref[i, 0], bounds_ref[i, 1] + 1, _seg_body, 0)

    @pl.when(i == NI - 1)
    def _head():
        p = p_ref[...]
        z = lax.dot_general(p, W1_ref[...], _NT,
                            preferred_element_type=jnp.float32) + b1_ref[...]
        z = jnp.where(z >= 0, z, 0.01 * z)
        z = lax.dot_general(z, W2_ref[...], _NT,
                            preferred_element_type=jnp.float32) + b2_ref[...]
        z = jnp.where(z >= 0, z, 0.01 * z)
        out_ref[...] = lax.dot_general(z, Wo_ref[...], _NT,
                                       preferred_element_type=jnp.float32) + bo_ref[...]


def kernel(x, adj, batch, n_nodes, Wg, bg, W1, b1, W2, b2, Wo, bo):
    del n_nodes
    batch3 = batch.reshape(NI, BI, 1)
    bounds = jnp.stack([batch[::BI], batch[BI - 1::BI]], axis=1)
    return pl.pallas_call(
        _fused_kernel,
        grid=(NI,),
        in_specs=[
            pl.BlockSpec(memory_space=pltpu.SMEM),
            pl.BlockSpec((N, D), lambda i: (0, 0)),
            pl.BlockSpec(memory_space=pltpu.MemorySpace.HBM),
            pl.BlockSpec((1, BI, 1), lambda i: (i, 0, 0)),
            pl.BlockSpec((D, H), lambda i: (0, 0)),
            pl.BlockSpec((1, H), lambda i: (0, 0)),
            pl.BlockSpec((H, H), lambda i: (0, 0)),
            pl.BlockSpec((1, H), lambda i: (0, 0)),
            pl.BlockSpec((H, H), lambda i: (0, 0)),
            pl.BlockSpec((1, H), lambda i: (0, 0)),
            pl.BlockSpec((O, H), lambda i: (0, 0)),
            pl.BlockSpec((1, O), lambda i: (0, 0)),
        ],
        out_specs=pl.BlockSpec((G, O), lambda i: (0, 0)),
        out_shape=jax.ShapeDtypeStruct((G, O), jnp.float32),
        scratch_shapes=[
            pltpu.VMEM((N, H), jnp.float32),
            pltpu.VMEM((G, H), jnp.float32),
            pltpu.VMEM((2, BI, N), jnp.float32),
            pltpu.SemaphoreType.DMA((2, SPLIT)),
        ],
    )(bounds, x, adj, batch3, Wg, bg, W1, b1, W2, b2, Wo, bo)
